# baseline (speedup 1.0000x reference)
import functools

import jax
import jax.numpy as jnp
from jax import lax
from jax.experimental import pallas as pl
from jax.experimental.pallas import tpu as pltpu
from jax.experimental.pallas import tpu_sc as plsc

ND = 50000
NE = 1600000
NR = 2
FIN = 128
FEFF = 16
FOUT = 64

ROWS = ND * NR
NSC = 16
CP_TILES = 10
CP_ROWS = ROWS // CP_TILES
CHUNK = 625
IDX_ROWS = NE // CHUNK
ROWS_PER_TILE = IDX_ROWS // NSC
ITERS = ROWS_PER_TILE


_EBLK = 320


def _idx_body(src_ref, dst_ref, et_ref, ia_ref, ib_ref):
    et = et_ref[...]
    ia_ref[...] = src_ref[...] * 2 + et
    ib_ref[...] = dst_ref[...] * 2 + et


def _make_indices(src, dst, et):
    bs = pl.BlockSpec((_EBLK, CHUNK), lambda i: (i, 0))
    out = jax.ShapeDtypeStruct((IDX_ROWS, CHUNK), jnp.int32)
    shp = (IDX_ROWS, CHUNK)
    return pl.pallas_call(
        _idx_body,
        grid=(IDX_ROWS // _EBLK,),
        in_specs=[bs, bs, bs],
        out_specs=[bs, bs],
        out_shape=[out, out],
    )(src.reshape(shp), dst.reshape(shp), et.reshape(shp))


_PBLK = 5000


def _proj_body(att_ref, feat_ref, cj_ref, basis_ref, out_ref):
    b0 = basis_ref[0]
    b1 = basis_ref[1]
    w0 = att_ref[0, 0] * b0 + att_ref[0, 1] * b1
    w1 = att_ref[1, 0] * b0 + att_ref[1, 1] * b1
    w = jnp.concatenate([w0, w1], axis=1)
    d = jnp.dot(feat_ref[...], w, preferred_element_type=jnp.float32)
    out_ref[...] = d * cj_ref[...]


def _project(feat, cj, att, basis):
    out = pl.pallas_call(
        _proj_body,
        grid=(ND // _PBLK,),
        in_specs=[
            pl.BlockSpec(memory_space=pltpu.SMEM),
            pl.BlockSpec((_PBLK, FIN), lambda i: (i, 0)),
            pl.BlockSpec((_PBLK, 1), lambda i: (i, 0)),
            pl.BlockSpec((NR, FIN, FEFF), lambda i: (0, 0, 0)),
        ],
        out_specs=pl.BlockSpec((_PBLK, NR * FEFF), lambda i: (i, 0)),
        out_shape=jax.ShapeDtypeStruct((ND, NR * FEFF), jnp.float32),
    )(att, feat, cj.reshape(ND, 1), basis)
    return out.reshape(ROWS, FEFF)


def _sc_message_passing(fd2, fs2, ia, ib, zrows):
    mesh = plsc.VectorSubcoreMesh(core_axis_name="c", subcore_axis_name="s")

    @functools.partial(
        pl.kernel,
        out_type=jax.ShapeDtypeStruct((2, ROWS, FEFF), jnp.float32),
        mesh=mesh,
        scratch_types=[
            pltpu.VMEM_SHARED((ROWS, FEFF), jnp.float32),
            pltpu.VMEM((2, CHUNK), jnp.int32),
            pltpu.VMEM((2, CHUNK), jnp.int32),
            pltpu.VMEM((2, CHUNK, FEFF), jnp.float32),
            pltpu.SemaphoreType.DMA,
            pltpu.SemaphoreType.DMA,
        ],
        compiler_params=pltpu.CompilerParams(use_tc_tiling_on_sc=False),
    )
    def k(fd2_hbm, fs2_hbm, ia_hbm, ib_hbm, z_hbm, out_hbm,
          acc, gbuf, sbuf, rows, gsem, ssem):
        c = lax.axis_index("c")
        s = lax.axis_index("s")

        @pl.when(s < CP_TILES)
        def _():
            pltpu.sync_copy(z_hbm, acc.at[pl.ds(s * CP_ROWS, CP_ROWS)])
        plsc.subcore_barrier()

        def run_dir(tbl_hbm, gidx_hbm, sidx_hbm):
            base = s * ROWS_PER_TILE

            def load_idx(cc, p):
                pltpu.sync_copy(gidx_hbm.at[base + cc], gbuf.at[p])
                pltpu.sync_copy(sidx_hbm.at[base + cc], sbuf.at[p])

            def gfire(p):
                pltpu.async_copy(tbl_hbm.at[gbuf.at[p]], rows.at[p], gsem)

            def gdrain(p):
                pltpu.make_async_copy(tbl_hbm.at[gbuf.at[p]], rows.at[p],
                                      gsem).wait()

            def sfire(p):
                pltpu.async_copy(rows.at[p], acc.at[sbuf.at[p]], ssem,
                                 add=True)

            def sdrain(p):
                pltpu.make_async_copy(rows.at[p], acc.at[sbuf.at[p]],
                                      ssem).wait()

            H = ITERS // 2
            load_idx(0, 0)
            gfire(0)

            @pl.loop(0, H)
            def _(h):
                c0 = 2 * h

                @pl.when(h > 0)
                def _():
                    sdrain(1)

                load_idx(c0 + 1, 1)
                gdrain(0)
                gfire(1)
                sfire(0)

                sdrain(0)

                @pl.when(h < H - 1)
                def _():
                    load_idx(c0 + 2, 0)

                gdrain(1)

                @pl.when(h < H - 1)
                def _():
                    gfire(0)

                sfire(1)

            sdrain(1)

        @pl.when(c == 0)
        def _():
            run_dir(fd2_hbm, ia_hbm, ib_hbm)

        @pl.when(c == 1)
        def _():
            run_dir(fs2_hbm, ib_hbm, ia_hbm)

        plsc.subcore_barrier()

        @pl.when(s < CP_TILES)
        def _():
            pltpu.sync_copy(acc.at[pl.ds(s * CP_ROWS, CP_ROWS)],
                            out_hbm.at[c, pl.ds(s * CP_ROWS, CP_ROWS)])

    return k(fd2, fs2, ia, ib, zrows)


_FBLK = 10000


def _fc_body(x_ref, ci_ref, wt_ref, b_ref, out_ref):
    d = jnp.dot(x_ref[...], wt_ref[...], preferred_element_type=jnp.float32)
    cb = ci_ref[...]
    d3 = d.reshape(_FBLK // 2, NR, FOUT)
    out_ref[...] = (d3 * cb[:, :, None]).reshape(_FBLK, FOUT) + b_ref[...]


def _fc(x, ci, wt, b):
    return pl.pallas_call(
        _fc_body,
        grid=(ROWS // _FBLK,),
        in_specs=[
            pl.BlockSpec((_FBLK, FEFF), lambda i: (i, 0)),
            pl.BlockSpec((_FBLK // 2, 1), lambda i: (i, 0)),
            pl.BlockSpec((FEFF, FOUT), lambda i: (0, 0)),
            pl.BlockSpec((1, FOUT), lambda i: (0, 0)),
        ],
        out_specs=pl.BlockSpec((_FBLK, FOUT), lambda i: (i, 0)),
        out_shape=jax.ShapeDtypeStruct((ROWS, FOUT), jnp.float32),
    )(x, ci, wt, b)


def kernel(drug_feat, dis_feat, edge_index, edge_type, cj_drug, ci_drug,
           cj_dis, ci_dis, att, basis, fc_W, fc_b):
    src = edge_index[0].astype(jnp.int32)
    dst = edge_index[1].astype(jnp.int32)
    et = edge_type.astype(jnp.int32)
    ia, ib = _make_indices(src, dst, et)
    fd2 = _project(drug_feat, cj_drug, att, basis)
    fs2 = _project(dis_feat, cj_dis, att, basis)

    zrows = jnp.zeros((CP_ROWS, FEFF), jnp.float32)
    acc = _sc_message_passing(fd2, fs2, ia, ib, zrows)

    wt = fc_W.T
    b = fc_b.reshape(1, FOUT)
    dis_out = _fc(acc[0], ci_dis.reshape(ND, 1), wt, b).reshape(ND, NR, FOUT)
    drug_out = _fc(acc[1], ci_drug.reshape(ND, 1), wt, b).reshape(ND, NR, FOUT)
    return drug_out, dis_out


# speedup vs baseline: 1.0053x; 1.0053x over previous
import functools

import jax
import jax.numpy as jnp
from jax import lax
from jax.experimental import pallas as pl
from jax.experimental.pallas import tpu as pltpu
from jax.experimental.pallas import tpu_sc as plsc

ND = 50000
NE = 1600000
NR = 2
FIN = 128
FEFF = 16
FOUT = 64

ROWS = ND * NR
NSC = 16
CP_TILES = 10
CP_ROWS = ROWS // CP_TILES
BATCH = 125
GRP = 5
CHUNK = BATCH * GRP
IDX_ROWS = NE // BATCH
ROWS_PER_TILE = IDX_ROWS // NSC
ITERS = ROWS_PER_TILE // GRP


_EBLK = 1600


def _idx_body(src_ref, dst_ref, et_ref, ia_ref, ib_ref):
    et = et_ref[...]
    ia_ref[...] = src_ref[...] * 2 + et
    ib_ref[...] = dst_ref[...] * 2 + et


def _make_indices(src, dst, et):
    bs = pl.BlockSpec((_EBLK, BATCH), lambda i: (i, 0))
    out = jax.ShapeDtypeStruct((IDX_ROWS, BATCH), jnp.int32)
    shp = (IDX_ROWS, BATCH)
    return pl.pallas_call(
        _idx_body,
        grid=(IDX_ROWS // _EBLK,),
        in_specs=[bs, bs, bs],
        out_specs=[bs, bs],
        out_shape=[out, out],
    )(src.reshape(shp), dst.reshape(shp), et.reshape(shp))


_PBLK = 5000


def _proj_body(att_ref, feat_ref, cj_ref, basis_ref, out_ref):
    b0 = basis_ref[0]
    b1 = basis_ref[1]
    w0 = att_ref[0, 0] * b0 + att_ref[0, 1] * b1
    w1 = att_ref[1, 0] * b0 + att_ref[1, 1] * b1
    w = jnp.concatenate([w0, w1], axis=1)
    d = jnp.dot(feat_ref[...], w, preferred_element_type=jnp.float32)
    out_ref[...] = d * cj_ref[...]


def _project(feat, cj, att, basis):
    out = pl.pallas_call(
        _proj_body,
        grid=(ND // _PBLK,),
        in_specs=[
            pl.BlockSpec(memory_space=pltpu.SMEM),
            pl.BlockSpec((_PBLK, FIN), lambda i: (i, 0)),
            pl.BlockSpec((_PBLK, 1), lambda i: (i, 0)),
            pl.BlockSpec((NR, FIN, FEFF), lambda i: (0, 0, 0)),
        ],
        out_specs=pl.BlockSpec((_PBLK, NR * FEFF), lambda i: (i, 0)),
        out_shape=jax.ShapeDtypeStruct((ND, NR * FEFF), jnp.float32),
    )(att, feat, cj.reshape(ND, 1), basis)
    return out.reshape(ROWS, FEFF)


def _sc_message_passing(fd2, fs2, ia, ib, zrows):
    mesh = plsc.VectorSubcoreMesh(core_axis_name="c", subcore_axis_name="s")

    @functools.partial(
        pl.kernel,
        out_type=jax.ShapeDtypeStruct((2, ROWS, FEFF), jnp.float32),
        mesh=mesh,
        scratch_types=[
            pltpu.VMEM_SHARED((ROWS, FEFF), jnp.float32),
            pltpu.VMEM((2, GRP, BATCH), jnp.int32),
            pltpu.VMEM((2, GRP, BATCH), jnp.int32),
            pltpu.VMEM((2, CHUNK, FEFF), jnp.float32),
            pltpu.SemaphoreType.DMA,
            pltpu.SemaphoreType.DMA,
        ],
        compiler_params=pltpu.CompilerParams(use_tc_tiling_on_sc=False),
    )
    def k(fd2_hbm, fs2_hbm, ia_hbm, ib_hbm, z_hbm, out_hbm,
          acc, gbuf, sbuf, rows, gsem, ssem):
        c = lax.axis_index("c")
        s = lax.axis_index("s")

        @pl.when(s < CP_TILES)
        def _():
            pltpu.sync_copy(z_hbm, acc.at[pl.ds(s * CP_ROWS, CP_ROWS)])
        plsc.subcore_barrier()

        def run_dir(tbl_hbm, gidx_hbm, sidx_hbm):
            base = s * ROWS_PER_TILE

            def load_idx(cc, p):
                pltpu.sync_copy(gidx_hbm.at[pl.ds(base + cc * GRP, GRP)],
                                gbuf.at[p])
                pltpu.sync_copy(sidx_hbm.at[pl.ds(base + cc * GRP, GRP)],
                                sbuf.at[p])

            def gfire(p):
                for j in range(GRP):
                    pltpu.async_copy(tbl_hbm.at[gbuf.at[p, j]],
                                     rows.at[p, pl.ds(j * BATCH, BATCH)],
                                     gsem)

            def gdrain(p):
                for j in range(GRP):
                    pltpu.make_async_copy(
                        tbl_hbm.at[gbuf.at[p, j]],
                        rows.at[p, pl.ds(j * BATCH, BATCH)], gsem).wait()

            def sfire(p):
                for j in range(GRP):
                    pltpu.async_copy(rows.at[p, pl.ds(j * BATCH, BATCH)],
                                     acc.at[sbuf.at[p, j]],
                                     ssem, add=True)

            def sdrain(p):
                for j in range(GRP):
                    pltpu.make_async_copy(
                        rows.at[p, pl.ds(j * BATCH, BATCH)],
                        acc.at[sbuf.at[p, j]], ssem).wait()

            H = ITERS // 2
            load_idx(0, 0)
            gfire(0)

            @pl.loop(0, H)
            def _(h):
                c0 = 2 * h

                @pl.when(h > 0)
                def _():
                    sdrain(1)

                load_idx(c0 + 1, 1)
                gdrain(0)
                gfire(1)
                sfire(0)

                sdrain(0)

                @pl.when(h < H - 1)
                def _():
                    load_idx(c0 + 2, 0)

                gdrain(1)

                @pl.when(h < H - 1)
                def _():
                    gfire(0)

                sfire(1)

            sdrain(1)

        @pl.when(c == 0)
        def _():
            run_dir(fd2_hbm, ia_hbm, ib_hbm)

        @pl.when(c == 1)
        def _():
            run_dir(fs2_hbm, ib_hbm, ia_hbm)

        plsc.subcore_barrier()

        @pl.when(s < CP_TILES)
        def _():
            pltpu.sync_copy(acc.at[pl.ds(s * CP_ROWS, CP_ROWS)],
                            out_hbm.at[c, pl.ds(s * CP_ROWS, CP_ROWS)])

    return k(fd2, fs2, ia, ib, zrows)


_FBLK = 10000


def _fc_body(x_ref, ci_ref, wt_ref, b_ref, out_ref):
    d = jnp.dot(x_ref[...], wt_ref[...], preferred_element_type=jnp.float32)
    cb = ci_ref[...]
    d3 = d.reshape(_FBLK // 2, NR, FOUT)
    out_ref[...] = (d3 * cb[:, :, None]).reshape(_FBLK, FOUT) + b_ref[...]


def _fc(x, ci, wt, b):
    return pl.pallas_call(
        _fc_body,
        grid=(ROWS // _FBLK,),
        in_specs=[
            pl.BlockSpec((_FBLK, FEFF), lambda i: (i, 0)),
            pl.BlockSpec((_FBLK // 2, 1), lambda i: (i, 0)),
            pl.BlockSpec((FEFF, FOUT), lambda i: (0, 0)),
            pl.BlockSpec((1, FOUT), lambda i: (0, 0)),
        ],
        out_specs=pl.BlockSpec((_FBLK, FOUT), lambda i: (i, 0)),
        out_shape=jax.ShapeDtypeStruct((ROWS, FOUT), jnp.float32),
    )(x, ci, wt, b)


def kernel(drug_feat, dis_feat, edge_index, edge_type, cj_drug, ci_drug,
           cj_dis, ci_dis, att, basis, fc_W, fc_b):
    src = edge_index[0].astype(jnp.int32)
    dst = edge_index[1].astype(jnp.int32)
    et = edge_type.astype(jnp.int32)
    ia, ib = _make_indices(src, dst, et)
    fd2 = _project(drug_feat, cj_drug, att, basis)
    fs2 = _project(dis_feat, cj_dis, att, basis)

    zrows = jnp.zeros((CP_ROWS, FEFF), jnp.float32)
    acc = _sc_message_passing(fd2, fs2, ia, ib, zrows)

    wt = fc_W.T
    b = fc_b.reshape(1, FOUT)
    dis_out = _fc(acc[0], ci_dis.reshape(ND, 1), wt, b).reshape(ND, NR, FOUT)
    drug_out = _fc(acc[1], ci_drug.reshape(ND, 1), wt, b).reshape(ND, NR, FOUT)
    return drug_out, dis_out
